# Initial kernel scaffold; baseline (speedup 1.0000x reference)
import jax
import jax.numpy as jnp
from jax.experimental import pallas as pl


def kernel(x, edge_index, W1, b1, W2, b2, Wl, bl):
    raise NotImplementedError("write your pallas kernel here")


# speedup vs baseline: 35.2178x; 35.2178x over previous
import functools

import jax
import jax.numpy as jnp
from jax import lax
from jax.experimental import pallas as pl
from jax.experimental.pallas import tpu as pltpu
from jax.experimental.pallas import tpu_sc as plsc

N = 10000
F = 128
H = 16
E = 320000

NC, NS = 2, 16
NW = NC * NS
IDXW = 128
KJ = 8
CH = KJ * IDXW
NSS = 10
TPW = NSS * CH
EP = NW * TPW
ACC_ROWS = 10240
RPT = ACC_ROWS // NS

_mesh = plsc.VectorSubcoreMesh(core_axis_name="c", subcore_axis_name="s")


@functools.partial(
    pl.kernel,
    mesh=_mesh,
    out_type=jax.ShapeDtypeStruct((NC, ACC_ROWS, H), jnp.float32),
    scratch_types=[
        pltpu.VMEM((KJ, IDXW), jnp.int32),
        pltpu.VMEM((KJ, IDXW), jnp.int32),
        pltpu.VMEM((CH, H), jnp.float32),
        pltpu.VMEM_SHARED((ACC_ROWS, H), jnp.float32),
        pltpu.SemaphoreType.DMA,
    ],
    compiler_params=pltpu.CompilerParams(use_tc_tiling_on_sc=False),
)
def _agg_sc(m_hbm, src_hbm, dst_hbm, out_hbm, sidx, didx, rows, acc, sem):
    c = lax.axis_index("c")
    s = lax.axis_index("s")
    wid = c * NS + s

    def _z(i, carry):
        rows[i, :] = jnp.zeros((H,), jnp.float32)
        return carry

    lax.fori_loop(0, RPT, _z, 0)
    pltpu.sync_copy(rows.at[pl.ds(0, RPT)], acc.at[pl.ds(s * RPT, RPT)])
    plsc.subcore_barrier()

    base = wid * (TPW // IDXW)

    def _step(ss, carry):
        r0 = base + ss * KJ
        pltpu.sync_copy(src_hbm.at[pl.ds(r0, KJ)], sidx)
        pltpu.sync_copy(dst_hbm.at[pl.ds(r0, KJ)], didx)
        cps = [
            pltpu.async_copy(m_hbm.at[sidx.at[j]], rows.at[pl.ds(j * IDXW, IDXW)], sem)
            for j in range(KJ)
        ]
        for cp in cps:
            cp.wait()
        for j in range(KJ):
            pltpu.sync_copy(rows.at[pl.ds(j * IDXW, IDXW)], acc.at[didx.at[j]], add=True)
        return carry

    lax.fori_loop(0, NSS, _step, 0)
    plsc.subcore_barrier()

    pltpu.sync_copy(acc.at[pl.ds(s * RPT, RPT)], rows.at[pl.ds(0, RPT)])
    pltpu.sync_copy(rows.at[pl.ds(0, RPT)], out_hbm.at[c].at[pl.ds(s * RPT, RPT)])


@functools.partial(
    pl.kernel,
    mesh=_mesh,
    out_type=jax.ShapeDtypeStruct((NC, ACC_ROWS, H), jnp.float32),
    scratch_types=[
        pltpu.VMEM((KJ, IDXW), jnp.int32),
        pltpu.VMEM((CH, H), jnp.float32),
        pltpu.VMEM_SHARED((ACC_ROWS, H), jnp.float32),
    ],
    compiler_params=pltpu.CompilerParams(use_tc_tiling_on_sc=False),
)
def _deg_sc(dst_hbm, out_hbm, didx, rows, acc):
    c = lax.axis_index("c")
    s = lax.axis_index("s")
    wid = c * NS + s

    def _z(i, carry):
        rows[i, :] = jnp.zeros((H,), jnp.float32)
        return carry

    lax.fori_loop(0, RPT, _z, 0)
    pltpu.sync_copy(rows.at[pl.ds(0, RPT)], acc.at[pl.ds(s * RPT, RPT)])
    plsc.subcore_barrier()

    def _o(i, carry):
        rows[i, :] = jnp.ones((H,), jnp.float32)
        return carry

    lax.fori_loop(0, IDXW, _o, 0)

    base = wid * (TPW // IDXW)

    def _step(ss, carry):
        r0 = base + ss * KJ
        pltpu.sync_copy(dst_hbm.at[pl.ds(r0, KJ)], didx)
        for j in range(KJ):
            pltpu.sync_copy(rows.at[pl.ds(0, IDXW)], acc.at[didx.at[j]], add=True)
        return carry

    lax.fori_loop(0, NSS, _step, 0)
    plsc.subcore_barrier()

    pltpu.sync_copy(acc.at[pl.ds(s * RPT, RPT)], rows.at[pl.ds(0, RPT)])
    pltpu.sync_copy(rows.at[pl.ds(0, RPT)], out_hbm.at[c].at[pl.ds(s * RPT, RPT)])


def _tc1_body(degp_ref, x_ref, w1_ref, m1_ref, dmat_ref):
    deg = degp_ref[0, :N, :] + degp_ref[1, :N, :] + 1.0
    d = lax.rsqrt(deg)
    u1 = jnp.dot(x_ref[...], w1_ref[...], preferred_element_type=jnp.float32)
    m1_ref[...] = d * u1
    dmat_ref[...] = d


def _tc2_body(p1_ref, m1_ref, dmat_ref, b1_ref, w2_ref, m2_ref):
    d = dmat_ref[...]
    h = d * (p1_ref[0, :N, :] + p1_ref[1, :N, :] + m1_ref[...]) + b1_ref[...]
    h = jnp.maximum(h, 0.0)
    m2_ref[...] = d * jnp.dot(h, w2_ref[...], preferred_element_type=jnp.float32)


def _tc3_body(p2_ref, m2_ref, dmat_ref, b2_ref, wl_ref, bl_ref, out_ref):
    d = dmat_ref[...]
    h = d * (p2_ref[0, :N, :] + p2_ref[1, :N, :] + m2_ref[...]) + b2_ref[...]
    h = jnp.maximum(h, 0.0)
    out_ref[...] = (
        jnp.dot(h, wl_ref[...], preferred_element_type=jnp.float32) + bl_ref[...]
    )


def kernel(x, edge_index, W1, b1, W2, b2, Wl, bl):
    src = edge_index[0]
    dst = edge_index[1]
    pad_s = jnp.zeros((EP - E,), jnp.int32)
    pad_d = jnp.full((EP - E,), N, jnp.int32)
    src2d = jnp.concatenate([src, pad_s]).reshape(EP // IDXW, IDXW)
    dst2d = jnp.concatenate([dst, pad_d]).reshape(EP // IDXW, IDXW)

    degp = _deg_sc(dst2d)

    m1, dmat = pl.pallas_call(
        _tc1_body,
        out_shape=(
            jax.ShapeDtypeStruct((N, H), jnp.float32),
            jax.ShapeDtypeStruct((N, H), jnp.float32),
        ),
    )(degp, x, W1)

    p1 = _agg_sc(m1, src2d, dst2d)

    m2 = pl.pallas_call(
        _tc2_body,
        out_shape=jax.ShapeDtypeStruct((N, H), jnp.float32),
    )(p1, m1, dmat, b1.reshape(1, H), W2)

    p2 = _agg_sc(m2, src2d, dst2d)

    out = pl.pallas_call(
        _tc3_body,
        out_shape=jax.ShapeDtypeStruct((N, 1), jnp.float32),
    )(p2, m2, dmat, b2.reshape(1, H), Wl, bl.reshape(1, 1))

    return out.reshape(-1)


# speedup vs baseline: 41.5137x; 1.1788x over previous
import functools

import jax
import jax.numpy as jnp
from jax import lax
from jax.experimental import pallas as pl
from jax.experimental.pallas import tpu as pltpu
from jax.experimental.pallas import tpu_sc as plsc

N = 10000
F = 128
H = 16
E = 320000

NC, NS = 2, 16
NW = NC * NS
IDXW = 128
KJ = 8
CH = KJ * IDXW
NSS = 10
TPW = NSS * CH
EP = NW * TPW
ACC_ROWS = 10240
RPT = ACC_ROWS // NS

_mesh = plsc.VectorSubcoreMesh(core_axis_name="c", subcore_axis_name="s")


@functools.partial(
    pl.kernel,
    mesh=_mesh,
    out_type=jax.ShapeDtypeStruct((NC, ACC_ROWS, H), jnp.float32),
    scratch_types=[
        pltpu.VMEM((TPW // IDXW, IDXW), jnp.int32),
        pltpu.VMEM((TPW // IDXW, IDXW), jnp.int32),
        pltpu.VMEM((CH, H), jnp.float32),
        pltpu.VMEM((CH, H), jnp.float32),
        pltpu.VMEM((RPT, H), jnp.float32),
        pltpu.VMEM_SHARED((ACC_ROWS, H), jnp.float32),
        pltpu.SemaphoreType.DMA,
        pltpu.SemaphoreType.DMA,
    ],
    compiler_params=pltpu.CompilerParams(use_tc_tiling_on_sc=False),
)
def _agg_sc(m_hbm, src_hbm, dst_hbm, out_hbm, sidx, didx, rows0, rows1, zbuf, acc, sem0, sem1):
    c = lax.axis_index("c")
    s = lax.axis_index("s")
    wid = c * NS + s
    rowsb = (rows0, rows1)
    sems = (sem0, sem1)

    base = wid * (TPW // IDXW)
    pltpu.sync_copy(src_hbm.at[pl.ds(base, TPW // IDXW)], sidx)
    pltpu.sync_copy(dst_hbm.at[pl.ds(base, TPW // IDXW)], didx)

    def fire(ss):
        buf = rowsb[ss % 2]
        return [
            pltpu.async_copy(
                m_hbm.at[sidx.at[ss * KJ + j]],
                buf.at[pl.ds(j * IDXW, IDXW)],
                sems[ss % 2],
            )
            for j in range(KJ)
        ]

    pend = {0: fire(0), 1: fire(1)}

    def _z(i, carry):
        zbuf[i, :] = jnp.zeros((H,), jnp.float32)
        return carry

    lax.fori_loop(0, RPT, _z, 0)
    pltpu.sync_copy(zbuf, acc.at[pl.ds(s * RPT, RPT)])
    plsc.subcore_barrier()

    for ss in range(NSS):
        p = ss % 2
        for cp in pend.pop(ss):
            cp.wait()
        buf = rowsb[p]
        for j in range(KJ):
            pltpu.sync_copy(
                buf.at[pl.ds(j * IDXW, IDXW)], acc.at[didx.at[ss * KJ + j]], add=True
            )
        if ss + 2 < NSS:
            pend[ss + 2] = fire(ss + 2)
    plsc.subcore_barrier()

    pltpu.sync_copy(acc.at[pl.ds(s * RPT, RPT)], zbuf)
    pltpu.sync_copy(zbuf, out_hbm.at[c].at[pl.ds(s * RPT, RPT)])


@functools.partial(
    pl.kernel,
    mesh=_mesh,
    out_type=jax.ShapeDtypeStruct((NC, ACC_ROWS, H), jnp.float32),
    scratch_types=[
        pltpu.VMEM((TPW // IDXW, IDXW), jnp.int32),
        pltpu.VMEM((RPT, H), jnp.float32),
        pltpu.VMEM_SHARED((ACC_ROWS, H), jnp.float32),
    ],
    compiler_params=pltpu.CompilerParams(use_tc_tiling_on_sc=False),
)
def _deg_sc(dst_hbm, out_hbm, didx, rows, acc):
    c = lax.axis_index("c")
    s = lax.axis_index("s")
    wid = c * NS + s

    base = wid * (TPW // IDXW)
    pltpu.sync_copy(dst_hbm.at[pl.ds(base, TPW // IDXW)], didx)

    def _z(i, carry):
        rows[i, :] = jnp.zeros((H,), jnp.float32)
        return carry

    lax.fori_loop(0, RPT, _z, 0)
    pltpu.sync_copy(rows, acc.at[pl.ds(s * RPT, RPT)])
    plsc.subcore_barrier()

    def _o(i, carry):
        rows[i, :] = jnp.ones((H,), jnp.float32)
        return carry

    lax.fori_loop(0, IDXW, _o, 0)

    def _step(r, carry):
        pltpu.sync_copy(rows.at[pl.ds(0, IDXW)], acc.at[didx.at[r]], add=True)
        return carry

    lax.fori_loop(0, TPW // IDXW, _step, 0)
    plsc.subcore_barrier()

    pltpu.sync_copy(acc.at[pl.ds(s * RPT, RPT)], rows)
    pltpu.sync_copy(rows, out_hbm.at[c].at[pl.ds(s * RPT, RPT)])


def _tc1_body(degp_ref, x_ref, w1_ref, m1_ref, dmat_ref):
    deg = degp_ref[0, :N, :] + degp_ref[1, :N, :] + 1.0
    d = lax.rsqrt(deg)
    u1 = jnp.dot(x_ref[...], w1_ref[...], preferred_element_type=jnp.float32)
    m1_ref[...] = d * u1
    dmat_ref[...] = d


def _tc2_body(p1_ref, m1_ref, dmat_ref, b1_ref, w2_ref, m2_ref):
    d = dmat_ref[...]
    h = d * (p1_ref[0, :N, :] + p1_ref[1, :N, :] + m1_ref[...]) + b1_ref[...]
    h = jnp.maximum(h, 0.0)
    m2_ref[...] = d * jnp.dot(h, w2_ref[...], preferred_element_type=jnp.float32)


def _tc3_body(p2_ref, m2_ref, dmat_ref, b2_ref, wl_ref, bl_ref, out_ref):
    d = dmat_ref[...]
    h = d * (p2_ref[0, :N, :] + p2_ref[1, :N, :] + m2_ref[...]) + b2_ref[...]
    h = jnp.maximum(h, 0.0)
    out_ref[...] = (
        jnp.dot(h, wl_ref[...], preferred_element_type=jnp.float32) + bl_ref[...]
    )


def kernel(x, edge_index, W1, b1, W2, b2, Wl, bl):
    src = edge_index[0]
    dst = edge_index[1]
    pad_s = jnp.zeros((EP - E,), jnp.int32)
    pad_d = jnp.full((EP - E,), N, jnp.int32)
    src2d = jnp.concatenate([src, pad_s]).reshape(EP // IDXW, IDXW)
    dst2d = jnp.concatenate([dst, pad_d]).reshape(EP // IDXW, IDXW)

    degp = _deg_sc(dst2d)

    m1, dmat = pl.pallas_call(
        _tc1_body,
        out_shape=(
            jax.ShapeDtypeStruct((N, H), jnp.float32),
            jax.ShapeDtypeStruct((N, H), jnp.float32),
        ),
    )(degp, x, W1)

    p1 = _agg_sc(m1, src2d, dst2d)

    m2 = pl.pallas_call(
        _tc2_body,
        out_shape=jax.ShapeDtypeStruct((N, H), jnp.float32),
    )(p1, m1, dmat, b1.reshape(1, H), W2)

    p2 = _agg_sc(m2, src2d, dst2d)

    out = pl.pallas_call(
        _tc3_body,
        out_shape=jax.ShapeDtypeStruct((N, 1), jnp.float32),
    )(p2, m2, dmat, b2.reshape(1, H), Wl, bl.reshape(1, 1))

    return out.reshape(-1)


# speedup vs baseline: 59.2995x; 1.4284x over previous
import functools

import jax
import jax.numpy as jnp
from jax import lax
from jax.experimental import pallas as pl
from jax.experimental.pallas import tpu as pltpu
from jax.experimental.pallas import tpu_sc as plsc

N = 10000
F = 128
H = 16
E = 320000

NC, NS = 2, 16
NW = NC * NS
IDXW = 128
KJ = 8
CH = KJ * IDXW
NSS = 10
TPW = NSS * CH
EP = NW * TPW
ACC_ROWS = 10240
RPT = ACC_ROWS // NS

_mesh = plsc.VectorSubcoreMesh(core_axis_name="c", subcore_axis_name="s")


@functools.partial(
    pl.kernel,
    mesh=_mesh,
    out_type=jax.ShapeDtypeStruct((NC, ACC_ROWS, H), jnp.float32),
    scratch_types=[
        pltpu.VMEM((TPW // IDXW, IDXW), jnp.int32),
        pltpu.VMEM((TPW // IDXW, IDXW), jnp.int32),
        pltpu.VMEM((CH, H), jnp.float32),
        pltpu.VMEM((CH, H), jnp.float32),
        pltpu.VMEM((RPT, H), jnp.float32),
        pltpu.VMEM_SHARED((ACC_ROWS, H), jnp.float32),
        pltpu.SemaphoreType.DMA,
        pltpu.SemaphoreType.DMA,
    ],
    compiler_params=pltpu.CompilerParams(use_tc_tiling_on_sc=False),
)
def _agg_sc(m_hbm, src_hbm, dst_hbm, out_hbm, sidx, didx, rows0, rows1, zbuf, acc, sem0, sem1):
    c = lax.axis_index("c")
    s = lax.axis_index("s")
    wid = c * NS + s
    rowsb = (rows0, rows1)
    sems = (sem0, sem1)

    base = wid * (TPW // IDXW)
    pltpu.sync_copy(src_hbm.at[pl.ds(base, TPW // IDXW)], sidx)
    pltpu.sync_copy(dst_hbm.at[pl.ds(base, TPW // IDXW)], didx)

    def fire(ss):
        buf = rowsb[ss % 2]
        return [
            pltpu.async_copy(
                m_hbm.at[sidx.at[ss * KJ + j]],
                buf.at[pl.ds(j * IDXW, IDXW)],
                sems[ss % 2],
            )
            for j in range(KJ)
        ]

    pend = {0: fire(0), 1: fire(1)}

    def _z(i, carry):
        zbuf[i, :] = jnp.zeros((H,), jnp.float32)
        return carry

    lax.fori_loop(0, RPT, _z, 0)
    pltpu.sync_copy(zbuf, acc.at[pl.ds(s * RPT, RPT)])
    plsc.subcore_barrier()

    for ss in range(NSS):
        p = ss % 2
        for cp in pend.pop(ss):
            cp.wait()
        buf = rowsb[p]
        for j in range(KJ):
            pltpu.sync_copy(
                buf.at[pl.ds(j * IDXW, IDXW)], acc.at[didx.at[ss * KJ + j]], add=True
            )
        if ss + 2 < NSS:
            pend[ss + 2] = fire(ss + 2)
    plsc.subcore_barrier()

    pltpu.sync_copy(acc.at[pl.ds(s * RPT, RPT)], zbuf)
    pltpu.sync_copy(zbuf, out_hbm.at[c].at[pl.ds(s * RPT, RPT)])


@functools.partial(
    pl.kernel,
    mesh=_mesh,
    out_type=jax.ShapeDtypeStruct((NC, ACC_ROWS, H), jnp.float32),
    scratch_types=[
        pltpu.VMEM((TPW // IDXW, IDXW), jnp.int32),
        pltpu.VMEM((RPT, H), jnp.float32),
        pltpu.VMEM_SHARED((ACC_ROWS, H), jnp.float32),
    ],
    compiler_params=pltpu.CompilerParams(use_tc_tiling_on_sc=False),
)
def _deg_sc(dst_hbm, out_hbm, didx, rows, acc):
    c = lax.axis_index("c")
    s = lax.axis_index("s")
    wid = c * NS + s

    base = wid * (TPW // IDXW)
    pltpu.sync_copy(dst_hbm.at[pl.ds(base, TPW // IDXW)], didx)

    def _z(i, carry):
        rows[i, :] = jnp.zeros((H,), jnp.float32)
        return carry

    lax.fori_loop(0, RPT, _z, 0)
    pltpu.sync_copy(rows, acc.at[pl.ds(s * RPT, RPT)])
    plsc.subcore_barrier()

    def _o(i, carry):
        rows[i, :] = jnp.ones((H,), jnp.float32)
        return carry

    lax.fori_loop(0, IDXW, _o, 0)

    def _step(r, carry):
        pltpu.sync_copy(rows.at[pl.ds(0, IDXW)], acc.at[didx.at[r]], add=True)
        return carry

    lax.fori_loop(0, TPW // IDXW, _step, 0)
    plsc.subcore_barrier()

    pltpu.sync_copy(acc.at[pl.ds(s * RPT, RPT)], rows)
    pltpu.sync_copy(rows, out_hbm.at[c].at[pl.ds(s * RPT, RPT)])


def _tc1_body(degp_ref, x_ref, w1_ref, m1_ref, dmat_ref):
    deg = degp_ref[0, :N, :] + degp_ref[1, :N, :] + 1.0
    d = lax.rsqrt(deg)
    u1 = jnp.dot(x_ref[...], w1_ref[...], preferred_element_type=jnp.float32)
    m1_ref[...] = d * u1
    dmat_ref[...] = d


def _tc2_body(p1_ref, m1_ref, dmat_ref, b1_ref, w2_ref, m2_ref):
    d = dmat_ref[...]
    h = d * (p1_ref[0, :N, :] + p1_ref[1, :N, :] + m1_ref[...]) + b1_ref[...]
    h = jnp.maximum(h, 0.0)
    m2_ref[...] = d * jnp.dot(h, w2_ref[...], preferred_element_type=jnp.float32)


def _tc3_body(p2_ref, m2_ref, dmat_ref, b2_ref, wl_ref, bl_ref, out_ref):
    d = dmat_ref[...]
    h = d * (p2_ref[0, :N, :] + p2_ref[1, :N, :] + m2_ref[...]) + b2_ref[...]
    h = jnp.maximum(h, 0.0)
    out_ref[...] = (
        jnp.dot(h, wl_ref[...], preferred_element_type=jnp.float32) + bl_ref[...]
    )


def kernel(x, edge_index, W1, b1, W2, b2, Wl, bl):
    src = edge_index[0]
    dst = edge_index[1]
    lanes = jnp.arange(EP - E, dtype=jnp.int32) % 128
    pad_s = lanes
    pad_d = N + lanes
    src2d = jnp.concatenate([src, pad_s]).reshape(EP // IDXW, IDXW)
    dst2d = jnp.concatenate([dst, pad_d]).reshape(EP // IDXW, IDXW)

    degp = _deg_sc(dst2d)

    m1, dmat = pl.pallas_call(
        _tc1_body,
        out_shape=(
            jax.ShapeDtypeStruct((N, H), jnp.float32),
            jax.ShapeDtypeStruct((N, H), jnp.float32),
        ),
    )(degp, x, W1)

    p1 = _agg_sc(m1, src2d, dst2d)

    m2 = pl.pallas_call(
        _tc2_body,
        out_shape=jax.ShapeDtypeStruct((N, H), jnp.float32),
    )(p1, m1, dmat, b1.reshape(1, H), W2)

    p2 = _agg_sc(m2, src2d, dst2d)

    out = pl.pallas_call(
        _tc3_body,
        out_shape=jax.ShapeDtypeStruct((N, 1), jnp.float32),
    )(p2, m2, dmat, b2.reshape(1, H), Wl, bl.reshape(1, 1))

    return out.reshape(-1)
